# Initial kernel scaffold; baseline (speedup 1.0000x reference)
import jax
import jax.numpy as jnp
from jax.experimental import pallas as pl


def kernel(x, table):
    raise NotImplementedError("write your pallas kernel here")


# speedup vs baseline: 4.6652x; 4.6652x over previous
import functools

import jax
import jax.numpy as jnp
from jax import lax
from jax.experimental import pallas as pl
from jax.experimental.pallas import tpu as pltpu
from jax.experimental.pallas import tpu_sc as plsc

_TP = 8
_W = 128


def kernel(x, table):
    B, L = x.shape
    V, E = table.shape
    n_chunks = B // _TP
    N = B * L

    info = plsc.get_sparse_core_info()
    n_workers = info.num_cores * info.num_subcores
    n_win = N // _W
    win_per_worker = n_win // n_workers

    idx = (
        x.astype(jnp.int32)
        .reshape(n_chunks, _TP, L)
        .transpose(1, 0, 2)
        .reshape(N)
    )
    rows_per_worker = N // n_workers

    mesh = plsc.VectorSubcoreMesh(core_axis_name="c", subcore_axis_name="s")

    @functools.partial(
        pl.kernel,
        mesh=mesh,
        out_type=jax.ShapeDtypeStruct((N, E), table.dtype),
        scratch_types=[
            pltpu.VMEM((rows_per_worker,), jnp.int32),
            pltpu.VMEM((_W, E), table.dtype),
            pltpu.SemaphoreType.DMA,
        ],
        compiler_params=pltpu.CompilerParams(use_tc_tiling_on_sc=False),
    )
    def gather_kernel(table_hbm, idx_hbm, out_hbm, idx_v, rows_v, sem):
        wid = lax.axis_index("s") * info.num_cores + lax.axis_index("c")
        base_row = wid * rows_per_worker
        pltpu.sync_copy(idx_hbm.at[pl.ds(base_row, rows_per_worker)], idx_v)

        @pl.loop(0, win_per_worker)
        def _(j):
            pltpu.async_copy(
                table_hbm.at[idx_v.at[pl.ds(j * _W, _W)]], rows_v, sem
            ).wait()
            pltpu.sync_copy(rows_v, out_hbm.at[pl.ds(base_row + j * _W, _W)])

    out = gather_kernel(table, idx)
    return out.reshape(_TP, n_chunks * L, E)


# speedup vs baseline: 5.3271x; 1.1419x over previous
import functools

import jax
import jax.numpy as jnp
from jax import lax
from jax.experimental import pallas as pl
from jax.experimental.pallas import tpu as pltpu
from jax.experimental.pallas import tpu_sc as plsc

_TP = 8
_W = 128
_C_WIN = 5


def kernel(x, table):
    B, L = x.shape
    V, E = table.shape
    n_chunks = B // _TP
    N = B * L

    info = plsc.get_sparse_core_info()
    n_workers = info.num_cores * info.num_subcores
    rows_per_worker = N // n_workers
    win_per_worker = rows_per_worker // _W
    chunk_rows = _C_WIN * _W
    chunks_per_worker = win_per_worker // _C_WIN

    idx = (
        x.astype(jnp.int32)
        .reshape(n_chunks, _TP, L)
        .transpose(1, 0, 2)
        .reshape(N)
    )

    mesh = plsc.VectorSubcoreMesh(core_axis_name="c", subcore_axis_name="s")

    @functools.partial(
        pl.kernel,
        mesh=mesh,
        out_type=jax.ShapeDtypeStruct((N, E), table.dtype),
        scratch_types=[
            pltpu.VMEM((rows_per_worker,), jnp.int32),
            pltpu.VMEM((2, chunk_rows, E), table.dtype),
            pltpu.SemaphoreType.DMA,
            pltpu.SemaphoreType.DMA,
            pltpu.SemaphoreType.DMA,
            pltpu.SemaphoreType.DMA,
        ],
        compiler_params=pltpu.CompilerParams(use_tc_tiling_on_sc=False),
    )
    def gather_kernel(
        table_hbm, idx_hbm, out_hbm, idx_v, rows_v, g0, g1, w0, w1
    ):
        gsem = (g0, g1)
        wsem = (w0, w1)
        wid = lax.axis_index("s") * info.num_cores + lax.axis_index("c")
        base_row = wid * rows_per_worker
        pltpu.sync_copy(idx_hbm.at[pl.ds(base_row, rows_per_worker)], idx_v)

        def fire_gathers(g):
            b = g % 2
            return [
                pltpu.async_copy(
                    table_hbm.at[
                        idx_v.at[pl.ds(g * chunk_rows + w * _W, _W)]
                    ],
                    rows_v.at[b, pl.ds(w * _W, _W)],
                    gsem[b],
                )
                for w in range(_C_WIN)
            ]

        def fire_writeback(g):
            b = g % 2
            return pltpu.async_copy(
                rows_v.at[b],
                out_hbm.at[pl.ds(base_row + g * chunk_rows, chunk_rows)],
                wsem[b],
            )

        gops = {}
        wops = {}
        for g in range(chunks_per_worker + 1):
            if g < chunks_per_worker:
                if g >= 2:
                    wops[g - 2].wait()
                gops[g] = fire_gathers(g)
            if g >= 1:
                for c in gops[g - 1]:
                    c.wait()
                wops[g - 1] = fire_writeback(g - 1)
        wops[chunks_per_worker - 2].wait()
        wops[chunks_per_worker - 1].wait()

    out = gather_kernel(table, idx)
    return out.reshape(_TP, n_chunks * L, E)
